# baseline (speedup 1.0000x reference)
import functools

import jax
import jax.numpy as jnp
from jax import lax
from jax.experimental import pallas as pl
from jax.experimental.pallas import tpu as pltpu
from jax.experimental.pallas import tpu_sc as plsc

VOCAB = 1000000
D = 64
B = 4096
L = 200
SCALE = 8.0

NC = 2
NS = 16
NW = NC * NS

TOTAL = B * L
IDX_MINOR = 128
XROWS = TOTAL // IDX_MINOR
XROWS_PER_W = XROWS // NW
CHUNK_XROWS = 4
CHUNK = CHUNK_XROWS * IDX_MINOR
NCHUNKS = XROWS_PER_W // CHUNK_XROWS
ROWS_PER_W = TOTAL // NW


def _embed_body(x_hbm, table_hbm, out_hbm, idx_v, rows_v, sem):
    c = lax.axis_index("c")
    s = lax.axis_index("s")
    wid = s * NC + c

    def chunk_body(ci, carry):
        r0 = wid * XROWS_PER_W + ci * CHUNK_XROWS
        pltpu.sync_copy(x_hbm.at[pl.ds(r0, CHUNK_XROWS)], idx_v)
        cps = [
            pltpu.async_copy(
                table_hbm.at[idx_v.at[j]],
                rows_v.at[pl.ds(j * IDX_MINOR, IDX_MINOR)],
                sem,
            )
            for j in range(CHUNK_XROWS)
        ]
        for cp in cps:
            cp.wait()

        def scale_body(r, carry2):
            for l in range(D // 16):
                v = rows_v[r, pl.ds(l * 16, 16)]
                rows_v[r, pl.ds(l * 16, 16)] = v * SCALE
            return carry2

        lax.fori_loop(0, CHUNK, scale_body, 0, unroll=2)
        pltpu.sync_copy(
            rows_v, out_hbm.at[pl.ds(wid * ROWS_PER_W + ci * CHUNK, CHUNK)]
        )
        return carry

    lax.fori_loop(0, NCHUNKS, chunk_body, 0)


@jax.jit
def _embed(x2d, table):
    mesh = plsc.VectorSubcoreMesh(
        core_axis_name="c", subcore_axis_name="s", num_cores=NC, num_subcores=NS
    )
    return pl.kernel(
        _embed_body,
        out_type=jax.ShapeDtypeStruct((TOTAL, D), jnp.float32),
        mesh=mesh,
        scratch_types=[
            pltpu.VMEM((CHUNK_XROWS, IDX_MINOR), jnp.int32),
            pltpu.VMEM((CHUNK, D), jnp.float32),
            pltpu.SemaphoreType.DMA,
        ],
        compiler_params=pltpu.CompilerParams(use_tc_tiling_on_sc=False),
    )(x2d, table)


def kernel(x, table):
    x2d = x.reshape(-1).astype(jnp.int32).reshape(XROWS, IDX_MINOR)
    out = _embed(x2d, table)
    return out.reshape(B, L, D)
